# Initial kernel scaffold; baseline (speedup 1.0000x reference)
import jax
import jax.numpy as jnp
from jax.experimental import pallas as pl


def kernel(x, edge_index, edge_attr, data_batch, Wl1, bl1, Wr1, br1, We1, att1, b1, Wl2, bl2, Wr2, br2, We2, att2, b2, fW1, fb1, fW2, fb2, fW3, fb3, fW4, fb4, fW5, fb5, fW6, fb6):
    raise NotImplementedError("write your pallas kernel here")


# speedup vs baseline: 14.4805x; 14.4805x over previous
import functools

import jax
import jax.numpy as jnp
from jax import lax
from jax.experimental import pallas as pl
from jax.experimental.pallas import tpu as pltpu
from jax.experimental.pallas import tpu_sc as plsc

NN = 10000
EE = 320000
NG = 16
NWORK = 32
EPT = EE // NWORK
CH = 80
NCH = EPT // CH
EPTC = EE // 16
SLABC = 160
NSLABC = EPTC // SLABC
SUBC = SLABC // CH
ZCH = 200
NZCH = NN // ZCH
WC = 128
BN = 1000
BE = 1000

_mesh = plsc.VectorSubcoreMesh(core_axis_name="c", subcore_axis_name="s")


def _sc_gather(W):
    @functools.partial(
        pl.kernel,
        mesh=_mesh,
        out_type=(
            jax.ShapeDtypeStruct((EE, W), jnp.float32),
            jax.ShapeDtypeStruct((EE, W), jnp.float32),
        ),
        scratch_types=[
            pltpu.VMEM((CH,), jnp.int32),
            pltpu.VMEM((CH,), jnp.int32),
            pltpu.VMEM((CH, W), jnp.float32),
            pltpu.VMEM((CH, W), jnp.float32),
            pltpu.SemaphoreType.DMA,
            pltpu.SemaphoreType.DMA,
        ],
    )
    def k(tl_hbm, tr_hbm, src_hbm, dst_hbm, xls_hbm, xrd_hbm,
          sidx, didx, bufa, bufb, sema, semb):
        c = lax.axis_index("c")
        s = lax.axis_index("s")
        wid = c * 16 + s
        base = wid * EPT

        def body(j, carry):
            eb = base + j * CH
            pltpu.sync_copy(src_hbm.at[pl.ds(eb, CH)], sidx)
            pltpu.sync_copy(dst_hbm.at[pl.ds(eb, CH)], didx)
            cpa = pltpu.async_copy(tl_hbm.at[sidx], bufa, sema)
            cpb = pltpu.async_copy(tr_hbm.at[didx], bufb, semb)
            cpa.wait()
            cpb.wait()
            pltpu.sync_copy(bufa, xls_hbm.at[pl.ds(eb, CH)])
            pltpu.sync_copy(bufb, xrd_hbm.at[pl.ds(eb, CH)])
            return carry

        lax.fori_loop(0, NCH, body, 0)

    return k


def _sc_scatter():
    @functools.partial(
        pl.kernel,
        mesh=_mesh,
        out_type=jax.ShapeDtypeStruct((2, NN, WC), jnp.float32),
        scratch_types=[
            pltpu.VMEM((CH,), jnp.int32),
            pltpu.VMEM((SLABC, WC), jnp.float32),
            pltpu.VMEM((ZCH, WC), jnp.float32),
            pltpu.VMEM_SHARED((NN, WC), jnp.float32),
        ],
    )
    def k(s_hbm, dst_hbm, z_hbm, a_hbm, didx, slab, zbuf, acc_sh):
        c = lax.axis_index("c")
        s = lax.axis_index("s")
        base_e = s * EPTC

        pltpu.sync_copy(z_hbm, zbuf)
        for jj in range((NZCH + 15) // 16):
            j = s + jj * 16

            @pl.when(j < NZCH)
            def _():
                pltpu.sync_copy(zbuf, acc_sh.at[pl.ds(j * ZCH, ZCH)])

        plsc.subcore_barrier()

        def body(sl, carry):
            eb = base_e + sl * SLABC
            pltpu.sync_copy(s_hbm.at[c, pl.ds(eb, SLABC)], slab)
            for i in range(SUBC):
                pltpu.sync_copy(dst_hbm.at[pl.ds(eb + i * CH, CH)], didx)
                pltpu.sync_copy(slab.at[pl.ds(i * CH, CH)],
                                acc_sh.at[didx], add=True)
            return carry

        lax.fori_loop(0, NSLABC, body, 0)
        plsc.subcore_barrier()

        for jj in range((NZCH + 15) // 16):
            j = s + jj * 16

            @pl.when(j < NZCH)
            def _():
                pltpu.sync_copy(acc_sh.at[pl.ds(j * ZCH, ZCH)], zbuf)
                pltpu.sync_copy(zbuf, a_hbm.at[c, pl.ds(j * ZCH, ZCH)])

    return k


def _leaky(v, ns):
    return jnp.where(v >= 0, v, ns * v)


def _tc_node_mm(x_ref, wl_ref, bl_ref, wr_ref, br_ref, xl_ref, xr_ref):
    xv = x_ref[...]
    xl_ref[...] = jnp.dot(xv, wl_ref[...],
                          preferred_element_type=jnp.float32, precision=lax.Precision.HIGHEST) + bl_ref[...]
    xr_ref[...] = jnp.dot(xv, wr_ref[...],
                          preferred_element_type=jnp.float32, precision=lax.Precision.HIGHEST) + br_ref[...]


def _node_mm(x, Wl, bl, Wr, br, Din, Dout):
    grid = NN // BN
    return pl.pallas_call(
        _tc_node_mm,
        grid=(grid,),
        in_specs=[
            pl.BlockSpec((BN, Din), lambda i: (i, 0)),
            pl.BlockSpec((Din, Dout), lambda i: (0, 0)),
            pl.BlockSpec((1, Dout), lambda i: (0, 0)),
            pl.BlockSpec((Din, Dout), lambda i: (0, 0)),
            pl.BlockSpec((1, Dout), lambda i: (0, 0)),
        ],
        out_specs=[
            pl.BlockSpec((BN, Dout), lambda i: (i, 0)),
            pl.BlockSpec((BN, Dout), lambda i: (i, 0)),
        ],
        out_shape=(
            jax.ShapeDtypeStruct((NN, Dout), jnp.float32),
            jax.ShapeDtypeStruct((NN, Dout), jnp.float32),
        ),
    )(x, Wl, bl.reshape(1, Dout), Wr, br.reshape(1, Dout))


def _edge_s(xls, xrd, ea, We, attf, H, W, WIN):
    def body(xls_ref, xrd_ref, ea_ref, we_ref, att_ref, s_ref):
        e = jnp.dot(ea_ref[...], we_ref[...], preferred_element_type=jnp.float32, precision=lax.Precision.HIGHEST)
        xls_v = xls_ref[...][:, :W]
        m = _leaky(xls_v + xrd_ref[...][:, :W] + e, 0.2)
        t = m * att_ref[...]
        parts = []
        exs = []
        for h in range(H):
            lh = jnp.sum(t[:, h * 64:(h + 1) * 64], axis=1, keepdims=True)
            exh = jnp.exp(lh)
            exs.append(exh)
            parts.append(xls_v[:, h * 64:(h + 1) * 64] * exh)
        pad = jnp.zeros((BE, 2 * WC - W - H), jnp.float32)
        full = jnp.concatenate(parts + exs + [pad], axis=1)
        s_ref[...] = jnp.stack([full[:, :WC], full[:, WC:]], axis=0)

    grid = EE // BE
    return pl.pallas_call(
        body,
        grid=(grid,),
        in_specs=[
            pl.BlockSpec((BE, WIN), lambda i: (i, 0)),
            pl.BlockSpec((BE, WIN), lambda i: (i, 0)),
            pl.BlockSpec((BE, 16), lambda i: (i, 0)),
            pl.BlockSpec((16, W), lambda i: (0, 0)),
            pl.BlockSpec((1, W), lambda i: (0, 0)),
        ],
        out_specs=pl.BlockSpec((2, BE, WC), lambda i: (0, i, 0)),
        out_shape=jax.ShapeDtypeStruct((2, EE, WC), jnp.float32),
    )(xls, xrd, ea, We, attf)


def _finalize_mm(A, b, Wl, bl, Wr, br, H, W, Dout):
    def body(a_ref, b_ref, wl_ref, bl_ref, wr_ref, br_ref, xl_ref, xr_ref):
        a = a_ref[...]
        af = jnp.concatenate([a[0], a[1]], axis=1)
        num = af[:, :W]
        den = af[:, W:W + H]
        denb = jnp.repeat(den, 64, axis=1)
        h = _leaky(num / (denb + 1e-16) + b_ref[...], 0.01)
        xl_ref[...] = jnp.dot(h, wl_ref[...],
                              preferred_element_type=jnp.float32, precision=lax.Precision.HIGHEST) + bl_ref[...]
        xr_ref[...] = jnp.dot(h, wr_ref[...],
                              preferred_element_type=jnp.float32, precision=lax.Precision.HIGHEST) + br_ref[...]

    grid = NN // BN
    return pl.pallas_call(
        body,
        grid=(grid,),
        in_specs=[
            pl.BlockSpec((2, BN, WC), lambda i: (0, i, 0)),
            pl.BlockSpec((1, W), lambda i: (0, 0)),
            pl.BlockSpec((W, Dout), lambda i: (0, 0)),
            pl.BlockSpec((1, Dout), lambda i: (0, 0)),
            pl.BlockSpec((W, Dout), lambda i: (0, 0)),
            pl.BlockSpec((1, Dout), lambda i: (0, 0)),
        ],
        out_specs=[
            pl.BlockSpec((BN, Dout), lambda i: (i, 0)),
            pl.BlockSpec((BN, Dout), lambda i: (i, 0)),
        ],
        out_shape=(
            jax.ShapeDtypeStruct((NN, Dout), jnp.float32),
            jax.ShapeDtypeStruct((NN, Dout), jnp.float32),
        ),
    )(A, b.reshape(1, W), Wl, bl.reshape(1, Dout), Wr, br.reshape(1, Dout))


def _finalize_head(A, b2, db3, ws):
    H, W = 3, 192

    def body(a_ref, b_ref, db_ref, fw1, fb1, fw2, fb2, fw3, fb3, fw4, fb4,
             fw5, fb5, fw6, fb6, out_ref, acc):
        i = pl.program_id(0)

        @pl.when(i == 0)
        def _():
            acc[...] = jnp.zeros((NG, W), jnp.float32)

        a = a_ref[...]
        af = jnp.concatenate([a[0], a[1]], axis=1)
        num = af[:, :W]
        den = af[:, W:W + H]
        denb = jnp.repeat(den, 64, axis=1)
        h = _leaky(num / (denb + 1e-16) + b_ref[...], 0.01)
        g = db_ref[...][0, 0, :]
        oh = (g[:, None] == lax.broadcasted_iota(jnp.int32, (1, NG), 1)
              ).astype(jnp.float32)
        acc[...] += lax.dot_general(oh, h, (((0,), (0,)), ((), ())),
                                    preferred_element_type=jnp.float32, precision=lax.Precision.HIGHEST)

        @pl.when(i == (NN // BN) - 1)
        def _():
            p = acc[...]
            p = _leaky(jnp.dot(p, fw1[...], preferred_element_type=jnp.float32, precision=lax.Precision.HIGHEST)
                       + fb1[...], 0.01)
            p = _leaky(jnp.dot(p, fw2[...], preferred_element_type=jnp.float32, precision=lax.Precision.HIGHEST)
                       + fb2[...], 0.01)
            p = _leaky(jnp.dot(p, fw3[...], preferred_element_type=jnp.float32, precision=lax.Precision.HIGHEST)
                       + fb3[...], 0.01)
            p = _leaky(jnp.dot(p, fw4[...], preferred_element_type=jnp.float32, precision=lax.Precision.HIGHEST)
                       + fb4[...], 0.01)
            p = jnp.dot(p, fw5[...], preferred_element_type=jnp.float32, precision=lax.Precision.HIGHEST) + fb5[...]
            p = jnp.dot(p, fw6[...], preferred_element_type=jnp.float32, precision=lax.Precision.HIGHEST) + fb6[...]
            out_ref[...] = p

    grid = NN // BN
    specs = [
        pl.BlockSpec((2, BN, WC), lambda i: (0, i, 0)),
        pl.BlockSpec((1, W), lambda i: (0, 0)),
        pl.BlockSpec((1, 1, BN), lambda i: (i, 0, 0)),
    ]
    for (wa, wb) in [(192, 64), (64, 32), (32, 16), (16, 8), (8, 1), (1, 1)]:
        specs.append(pl.BlockSpec((wa, wb), lambda i: (0, 0)))
        specs.append(pl.BlockSpec((1, wb), lambda i: (0, 0)))
    args = [A, b2.reshape(1, W), db3]
    for (fw, fb) in ws:
        args.append(fw)
        args.append(fb.reshape(1, fb.shape[0]))
    return pl.pallas_call(
        body,
        grid=(grid,),
        in_specs=specs,
        out_specs=pl.BlockSpec((NG, 1), lambda i: (0, 0)),
        out_shape=jax.ShapeDtypeStruct((NG, 1), jnp.float32),
        scratch_shapes=[pltpu.VMEM((NG, W), jnp.float32)],
    )(*args)


def kernel(x, edge_index, edge_attr, data_batch, Wl1, bl1, Wr1, br1, We1,
           att1, b1, Wl2, bl2, Wr2, br2, We2, att2, b2, fW1, fb1, fW2, fb2,
           fW3, fb3, fW4, fb4, fW5, fb5, fW6, fb6):
    src1 = edge_index[0]
    dst1 = edge_index[1]
    z128 = jnp.zeros((ZCH, WC), jnp.float32)
    db3 = data_batch.reshape(NN // BN, 1, BN)
    Wl2p = jnp.pad(Wl2, ((0, 0), (0, 64)))
    Wr2p = jnp.pad(Wr2, ((0, 0), (0, 64)))
    bl2p = jnp.pad(bl2, (0, 64))
    br2p = jnp.pad(br2, (0, 64))

    xl1, xr1 = _node_mm(x, Wl1, bl1, Wr1, br1, 128, 128)
    xls1, xrd1 = _sc_gather(128)(xl1, xr1, src1, dst1)
    S1 = _edge_s(xls1, xrd1, edge_attr, We1, att1.reshape(1, 128), 2, 128, 128)
    A1 = _sc_scatter()(S1, dst1, z128)

    xl2, xr2 = _finalize_mm(A1, b1, Wl2p, bl2p, Wr2p, br2p, 2, 128, 256)

    xls2, xrd2 = _sc_gather(256)(xl2, xr2, src1, dst1)
    S2 = _edge_s(xls2, xrd2, edge_attr, We2, att2.reshape(1, 192), 3, 192, 256)
    A2 = _sc_scatter()(S2, dst1, z128)

    ws = [(fW1, fb1), (fW2, fb2), (fW3, fb3), (fW4, fb4), (fW5, fb5), (fW6, fb6)]
    return _finalize_head(A2, b2, db3, ws)
